# Initial kernel scaffold; baseline (speedup 1.0000x reference)
import jax
import jax.numpy as jnp
from jax.experimental import pallas as pl


def kernel(a, b, emb_table, ord_w):
    raise NotImplementedError("write your pallas kernel here")


# speedup vs baseline: 13.3239x; 13.3239x over previous
import jax
import jax.numpy as jnp
from jax.experimental import pallas as pl
from jax.experimental.pallas import tpu as pltpu

A_DIM = 512
T_DIM = 2048
STEPS = 16
N_ARR = 16
GROUPS = 32
EMB_D = 126


def _scan_body(x_ref, vals_ref, flats_ref, rowsq_ref):
    x = x_ref[...]
    rowsq_ref[...] = jnp.sum(x * x, axis=1, keepdims=True)

    col_iota2 = jax.lax.broadcasted_iota(jnp.int32, (A_DIM, T_DIM), 1)
    row_iota = jax.lax.broadcasted_iota(jnp.int32, (A_DIM, 1), 0)
    col_iota_row = jax.lax.broadcasted_iota(jnp.int32, (1, T_DIM), 1)
    step_iota = jax.lax.broadcasted_iota(jnp.int32, (STEPS, 1), 0)

    rowmax = jnp.max(x, axis=1, keepdims=True)
    rowarg = jnp.min(
        jnp.where(x == rowmax, col_iota2, T_DIM), axis=1, keepdims=True)

    vals0 = jnp.zeros((STEPS, 1), jnp.float32)
    flats0 = jnp.full((STEPS, 1), -1, jnp.int32)

    def step(k, carry):
        rowmax, rowarg, vals, flats = carry
        m = jnp.max(rowmax)
        r = jnp.min(jnp.where(rowmax == m, row_iota, A_DIM))
        c = jnp.sum(jnp.where(row_iota == r, rowarg, 0))
        flat = r * T_DIM + c
        vals = jnp.where(step_iota == k, m, vals)
        flats = jnp.where(step_iota == k, flat, flats)
        row = x_ref[pl.ds(r, 1), :]
        rmask = jnp.zeros((1, T_DIM), jnp.bool_)
        for k2 in range(STEPS):
            fk = flats[k2, 0]
            rk = fk >> 11
            ck = fk & (T_DIM - 1)
            rmask = rmask | ((rk == r) & (col_iota_row == ck))
        row_mod = jnp.where(rmask, 0.0, row)
        new_rmax = jnp.max(row_mod)
        new_rarg = jnp.min(
            jnp.where(row_mod == new_rmax, col_iota_row, T_DIM))
        rowmax = jnp.where(row_iota == r, new_rmax, rowmax)
        rowarg = jnp.where(row_iota == r, new_rarg, rowarg)
        return rowmax, rowarg, vals, flats

    _, _, vals, flats = jax.lax.fori_loop(
        0, STEPS, step, (rowmax, rowarg, vals0, flats0))
    vals_ref[...] = vals
    flats_ref[...] = flats


def _assemble_body(vals_ref, flats_ref, rowsq_ref, emb_ref, w_ref, out_ref):
    P = N_ARR * STEPS
    vals = vals_ref[...]
    flats = flats_ref[...]
    atom = flats >> 11
    t = flats & (T_DIM - 1)
    pos_idx = jnp.where(vals > 0, t,
                        jnp.where(vals == 0, 0, jnp.where(t != 0, 0, 1)))
    atom_idx = jnp.where(vals > 0, atom,
                         jnp.where(vals == 0, 0, jnp.where(atom != 0, 0, 1)))
    pos = pos_idx.astype(jnp.float32) * (20.0 / (T_DIM - 1))

    a_iota = jax.lax.broadcasted_iota(jnp.int32, (1, A_DIM), 1)
    onehot = (atom_idx == a_iota).astype(jnp.float32)
    emb_rows = jax.lax.dot_general(
        onehot, emb_ref[...], (((1,), (0,)), ((), ())),
        preferred_element_type=jnp.float32)

    rows = jnp.concatenate([pos, vals, emb_rows], axis=1)
    keys_col = jax.lax.dot_general(
        rows, w_ref[...], (((1,), (0,)), ((), ())),
        preferred_element_type=jnp.float32)
    keys_row = jax.lax.dot_general(
        w_ref[...], rows, (((0,), (1,)), ((), ())),
        preferred_element_type=jnp.float32)

    p_iota = jax.lax.broadcasted_iota(jnp.int32, (P, 1), 0)
    q_iota = jax.lax.broadcasted_iota(jnp.int32, (1, P), 1)
    same = (p_iota >> 4) == (q_iota >> 4)
    less = (keys_col < keys_row) | ((keys_col == keys_row) & (p_iota < q_iota))
    rank_row = jnp.sum((same & less).astype(jnp.int32),
                       axis=0, keepdims=True)
    perm = (same & (rank_row == (p_iota & 15))).astype(jnp.float32)
    sorted_rows = jax.lax.dot_general(
        perm, rows, (((1,), (0,)), ((), ())),
        preferred_element_type=jnp.float32)

    half = P // 2
    diff = sorted_rows[:half, :] - sorted_rows[half:, :]
    mse = jnp.sum(diff * diff) / float(half * 128)

    rowsq = rowsq_ref[...]
    r_iota = jax.lax.broadcasted_iota(jnp.int32, (A_DIM, 1), 0)
    g_iota = jax.lax.broadcasted_iota(jnp.int32, (1, GROUPS), 1)
    gmat = ((r_iota >> 4) == g_iota).astype(jnp.float32)
    gs = jax.lax.dot_general(
        rowsq, gmat, (((1,), (0,)), ((), ())),
        preferred_element_type=jnp.float32)

    s_iota = jax.lax.broadcasted_iota(jnp.int32, (N_ARR, 1), 0)
    sel = ((q_iota >> 4) == s_iota).astype(jnp.float32)
    grp = flats >> 15
    contrib = ((grp == g_iota).astype(jnp.float32)) * (vals * vals)
    removed = jax.lax.dot_general(
        sel, contrib, (((1,), (0,)), ((), ())),
        preferred_element_type=jnp.float32)

    resid = jnp.maximum(gs - removed, 0.0)
    norm = jnp.sqrt(resid)
    nh = N_ARR // 2
    nmean = jnp.sum(jnp.abs(norm[:nh, :] - norm[nh:, :])) / float(nh * GROUPS)

    out_ref[...] = jnp.full((1, 1), mse + nmean, jnp.float32)


def kernel(a, b, emb_table, ord_w):
    x = jnp.concatenate([a, b], axis=0)

    vals, flats, rowsq = pl.pallas_call(
        _scan_body,
        grid=(N_ARR,),
        in_specs=[pl.BlockSpec((None, A_DIM, T_DIM), lambda i: (i, 0, 0))],
        out_specs=[
            pl.BlockSpec((None, STEPS, 1), lambda i: (i, 0, 0)),
            pl.BlockSpec((None, STEPS, 1), lambda i: (i, 0, 0)),
            pl.BlockSpec((None, A_DIM, 1), lambda i: (i, 0, 0)),
        ],
        out_shape=[
            jax.ShapeDtypeStruct((N_ARR, STEPS, 1), jnp.float32),
            jax.ShapeDtypeStruct((N_ARR, STEPS, 1), jnp.int32),
            jax.ShapeDtypeStruct((N_ARR, A_DIM, 1), jnp.float32),
        ],
    )(x)

    out = pl.pallas_call(
        _assemble_body,
        out_shape=jax.ShapeDtypeStruct((1, 1), jnp.float32),
    )(
        vals.reshape(N_ARR * STEPS, 1),
        flats.reshape(N_ARR * STEPS, 1),
        rowsq.reshape(N_ARR, A_DIM),
        emb_table,
        ord_w.reshape(128, 1),
    )
    return out[0, 0]
